# baseline (speedup 1.0000x reference)
import functools

import jax
import jax.numpy as jnp
from jax import lax
from jax.experimental import pallas as pl
from jax.experimental.pallas import tpu as pltpu
from jax.experimental.pallas import tpu_sc as plsc

_B, _S = 16384, 50
_D = 32
_TOTAL = _B * _S
_NW = 32
_CHUNK = 128
_NCHUNKS = _TOTAL // _CHUNK
_PER_W = _NCHUNKS // _NW
_K = 10
_SUPER = _K * _CHUNK
_NSUPER = _PER_W // _K


def _body(tok_hbm, table_hbm, out_hbm, idx_v, buf_a, buf_b, g_a, g_b, st_a, st_b):
    cid = lax.axis_index("c")
    sid = lax.axis_index("s")
    wid = sid * 2 + cid
    wrow = wid * (_PER_W * _CHUNK)
    pltpu.sync_copy(tok_hbm.at[pl.ds(wid * _PER_W, _PER_W)], idx_v)

    def fire(s, buf, sem):
        cps = []
        for k in range(_K):
            cps.append(pltpu.async_copy(
                table_hbm.at[idx_v.at[s * _K + k]],
                buf.at[pl.ds(k * _CHUNK, _CHUNK)], sem))
        return cps

    def drain(buf, sem):
        for k in range(_K):
            pltpu.make_async_copy(
                table_hbm.at[idx_v.at[k]],
                buf.at[pl.ds(k * _CHUNK, _CHUNK)], sem).wait()

    def store(s, buf, sem):
        pltpu.async_copy(buf, out_hbm.at[pl.ds(wrow + s * _SUPER, _SUPER)], sem)

    def wait_store(buf, sem):
        pltpu.make_async_copy(buf, out_hbm.at[pl.ds(wrow, _SUPER)], sem).wait()

    fire(0, buf_a, g_a)

    def step(p, carry):
        s0 = 2 * p
        @pl.when(p > 0)
        def _():
            wait_store(buf_b, st_b)
        fire(s0 + 1, buf_b, g_b)
        drain(buf_a, g_a)
        store(s0, buf_a, st_a)
        @pl.when(p < _NSUPER // 2 - 1)
        def _():
            wait_store(buf_a, st_a)
            fire(s0 + 2, buf_a, g_a)
        drain(buf_b, g_b)
        store(s0 + 1, buf_b, st_b)
        return carry

    lax.fori_loop(0, _NSUPER // 2, step, 0)
    wait_store(buf_a, st_a)
    wait_store(buf_b, st_b)


@jax.jit
def _embed(tok2d, weight):
    mesh = plsc.VectorSubcoreMesh(core_axis_name="c", subcore_axis_name="s")
    kern = functools.partial(
        pl.kernel,
        mesh=mesh,
        out_type=jax.ShapeDtypeStruct((_TOTAL, _D), jnp.float32),
        scratch_types=[
            pltpu.VMEM((_PER_W, _CHUNK), jnp.int32),
            pltpu.VMEM((_SUPER, _D), jnp.float32),
            pltpu.VMEM((_SUPER, _D), jnp.float32),
            pltpu.SemaphoreType.DMA,
            pltpu.SemaphoreType.DMA,
            pltpu.SemaphoreType.DMA,
            pltpu.SemaphoreType.DMA,
        ],
        compiler_params=pltpu.CompilerParams(use_tc_tiling_on_sc=False),
    )(_body)
    return kern(tok2d, weight)


def kernel(token_ids, weight):
    tok = token_ids.reshape(_NCHUNKS, _CHUNK).astype(jnp.int32)
    out = _embed(tok, weight)
    return out.reshape(_B, _S, _D)


# speedup vs baseline: 1.4399x; 1.4399x over previous
import functools

import jax
import jax.numpy as jnp
from jax import lax
from jax.experimental import pallas as pl
from jax.experimental.pallas import tpu as pltpu
from jax.experimental.pallas import tpu_sc as plsc

_B, _S = 16384, 50
_D = 32
_NW = 32
_CPU = 4
_UN = _CPU * 128
_NUNITS = _S * (128 // _CPU)
_PER_W = _NUNITS // _NW
_CBLK = 128 // _CPU


def _transpose_unit(g, st):
    iota = lax.broadcasted_iota(jnp.int32, (16,), 0)
    cols = [jnp.full((16,), d, jnp.int32) for d in range(_D)]

    def tj(j, carry):
        cb = j // 8
        k = j % 8
        row = cb * 128 + k * 16 + iota
        for d in range(_D):
            vals = plsc.load_gather(g, [row, cols[d]])
            st[d // 8, cb, d % 8, pl.ds(k * 16, 16)] = vals
        return carry

    lax.fori_loop(0, _CPU * 8, tj, 0)


def _body(tok_hbm, table_hbm, out_hbm, idx_a, idx_b, g_a, g_b, st,
          gs_a, gs_b, st_sem):
    cid = lax.axis_index("c")
    sid = lax.axis_index("s")
    wid = sid * 2 + cid

    def fire(t, idx_v, g, gsem):
        u = wid * _PER_W + t
        s = u // _CBLK
        cb = u % _CBLK
        pltpu.sync_copy(tok_hbm.at[s, pl.ds(cb * _CPU, _CPU)], idx_v)
        for c in range(_CPU):
            pltpu.async_copy(table_hbm.at[idx_v.at[c]],
                             g.at[pl.ds(c * 128, 128)], gsem)

    def drain(idx_v, g, gsem):
        for c in range(_CPU):
            pltpu.make_async_copy(table_hbm.at[idx_v.at[c]],
                                  g.at[pl.ds(c * 128, 128)], gsem).wait()

    def store(t):
        u = wid * _PER_W + t
        s = u // _CBLK
        cb = u % _CBLK
        for r in range(4):
            pltpu.async_copy(st.at[r],
                             out_hbm.at[s, r, pl.ds(cb * _CPU, _CPU)], st_sem)

    def wait_store():
        for r in range(4):
            pltpu.make_async_copy(st.at[r],
                                  out_hbm.at[0, r, pl.ds(0, _CPU)],
                                  st_sem).wait()

    fire(0, idx_a, g_a, gs_a)

    def step(p, carry):
        t0 = 2 * p
        fire(t0 + 1, idx_b, g_b, gs_b)
        drain(idx_a, g_a, gs_a)
        @pl.when(p > 0)
        def _():
            wait_store()
        _transpose_unit(g_a, st)
        store(t0)
        @pl.when(p < _PER_W // 2 - 1)
        def _():
            fire(t0 + 2, idx_a, g_a, gs_a)
        drain(idx_b, g_b, gs_b)
        wait_store()
        _transpose_unit(g_b, st)
        store(t0 + 1)
        return carry

    lax.fori_loop(0, _PER_W // 2, step, 0)
    wait_store()


@jax.jit
def _embed(tok3, weight):
    mesh = plsc.VectorSubcoreMesh(core_axis_name="c", subcore_axis_name="s")
    kern = functools.partial(
        pl.kernel,
        mesh=mesh,
        out_type=jax.ShapeDtypeStruct((_S, 4, 128, 8, 128), jnp.float32),
        scratch_types=[
            pltpu.VMEM((_CPU, 128), jnp.int32),
            pltpu.VMEM((_CPU, 128), jnp.int32),
            pltpu.VMEM((_UN, _D), jnp.float32),
            pltpu.VMEM((_UN, _D), jnp.float32),
            pltpu.VMEM((4, _CPU, 8, 128), jnp.float32),
            pltpu.SemaphoreType.DMA,
            pltpu.SemaphoreType.DMA,
            pltpu.SemaphoreType.DMA,
        ],
        compiler_params=pltpu.CompilerParams(
            use_tc_tiling_on_sc=False, needs_layout_passes=False),
    )(_body)
    return kern(tok3, weight)


def kernel(token_ids, weight):
    tok3 = token_ids.T.reshape(_S, 128, 128).astype(jnp.int32)
    out5 = _embed(tok3, weight)
    return out5.transpose((2, 4, 0, 1, 3)).reshape(_B, _S, _D)


# speedup vs baseline: 1.6205x; 1.1255x over previous
import functools

import jax
import jax.numpy as jnp
from jax import lax
from jax.experimental import pallas as pl
from jax.experimental.pallas import tpu as pltpu
from jax.experimental.pallas import tpu_sc as plsc

_B, _S = 16384, 50
_D = 32
_NW = 32
_CPU = 4
_UN = _CPU * 128
_NUNITS = _S * (128 // _CPU)
_PER_W = _NUNITS // _NW
_CBLK = 128 // _CPU
_RPL = _CPU * 1024


def _transpose_unit(g, st):
    d16 = lax.broadcasted_iota(jnp.int32, (16,), 0)
    c_lo = (d16 // 8) * _RPL + (d16 % 8) * 128
    c_hi = c_lo + 2 * _RPL

    def tj(jo, carry):
        cb = jo // 16
        k8 = jo % 16
        row0 = cb * 128 + k8 * 8
        off0 = cb * 1024 + k8 * 8
        for u in range(8):
            j = row0 + u
            off = off0 + u
            vlo = g[j, pl.ds(0, 16)]
            vhi = g[j, pl.ds(16, 16)]
            plsc.store_scatter(st, [c_lo + off], vlo)
            plsc.store_scatter(st, [c_hi + off], vhi)
        return carry

    lax.fori_loop(0, _CPU * 16, tj, 0)


def _body(tok_hbm, table_hbm, out_hbm, idx_a, idx_b, g_a, g_b, st,
          gs_a, gs_b, st_sem):
    cid = lax.axis_index("c")
    sid = lax.axis_index("s")
    wid = sid * 2 + cid

    def fire(t, idx_v, g, gsem):
        u = wid * _PER_W + t
        s = u // _CBLK
        cb = u % _CBLK
        pltpu.sync_copy(tok_hbm.at[s, pl.ds(cb * _CPU, _CPU)], idx_v)
        for c in range(_CPU):
            pltpu.async_copy(table_hbm.at[idx_v.at[c]],
                             g.at[pl.ds(c * 128, 128)], gsem)

    def drain(idx_v, g, gsem):
        for c in range(_CPU):
            pltpu.make_async_copy(table_hbm.at[idx_v.at[c]],
                                  g.at[pl.ds(c * 128, 128)], gsem).wait()

    def store(t):
        u = wid * _PER_W + t
        s = u // _CBLK
        cb = u % _CBLK
        for r in range(4):
            pltpu.async_copy(st.at[pl.ds(r * _RPL, _RPL)],
                             out_hbm.at[s, r, pl.ds(cb * _RPL, _RPL)], st_sem)

    def wait_store():
        for r in range(4):
            pltpu.make_async_copy(st.at[pl.ds(r * _RPL, _RPL)],
                                  out_hbm.at[0, r, pl.ds(0, _RPL)],
                                  st_sem).wait()

    fire(0, idx_a, g_a, gs_a)

    def step(p, carry):
        t0 = 2 * p
        fire(t0 + 1, idx_b, g_b, gs_b)
        drain(idx_a, g_a, gs_a)
        @pl.when(p > 0)
        def _():
            wait_store()
        _transpose_unit(g_a, st)
        store(t0)
        @pl.when(p < _PER_W // 2 - 1)
        def _():
            fire(t0 + 2, idx_a, g_a, gs_a)
        drain(idx_b, g_b, gs_b)
        wait_store()
        _transpose_unit(g_b, st)
        store(t0 + 1)
        return carry

    lax.fori_loop(0, _PER_W // 2, step, 0)
    wait_store()


@jax.jit
def _embed(tok3, weight):
    mesh = plsc.VectorSubcoreMesh(core_axis_name="c", subcore_axis_name="s")
    kern = functools.partial(
        pl.kernel,
        mesh=mesh,
        out_type=jax.ShapeDtypeStruct((_S, 4, 128 * 1024), jnp.float32),
        scratch_types=[
            pltpu.VMEM((_CPU, 128), jnp.int32),
            pltpu.VMEM((_CPU, 128), jnp.int32),
            pltpu.VMEM((_UN, _D), jnp.float32),
            pltpu.VMEM((_UN, _D), jnp.float32),
            pltpu.VMEM((4 * _RPL,), jnp.float32),
            pltpu.SemaphoreType.DMA,
            pltpu.SemaphoreType.DMA,
            pltpu.SemaphoreType.DMA,
        ],
        compiler_params=pltpu.CompilerParams(
            use_tc_tiling_on_sc=False, needs_layout_passes=False),
    )(_body)
    return kern(tok3, weight)


def kernel(token_ids, weight):
    tok3 = token_ids.T.reshape(_S, 128, 128).astype(jnp.int32)
    out5 = _embed(tok3, weight).reshape(_S, 4, 128, 8, 128)
    return out5.transpose((2, 4, 0, 1, 3)).reshape(_B, _S, _D)


# speedup vs baseline: 2.4582x; 1.5169x over previous
import functools

import jax
import jax.numpy as jnp
from jax import lax
from jax.experimental import pallas as pl
from jax.experimental.pallas import tpu as pltpu
from jax.experimental.pallas import tpu_sc as plsc

_B, _S = 16384, 50
_D = 32
_NW = 32
_CPU = 4
_UN = _CPU * 128
_NUNITS = _S * (128 // _CPU)
_PER_W = _NUNITS // _NW
_CBLK = 128 // _CPU
_RPL = _CPU * 1024


def _transpose_unit(g, st):
    d16 = lax.broadcasted_iota(jnp.int32, (16,), 0)
    i1_lo = d16 // 8
    i1_hi = i1_lo + 2
    i2 = d16 % 8

    def tj(jo, carry):
        cb = jo // 16
        k8 = jo % 16
        row0 = cb * 128 + k8 * 8
        cbv = jnp.full((16,), cb, jnp.int32)
        for u in range(8):
            j = row0 + u
            nnv = jnp.full((16,), k8 * 8 + u, jnp.int32)
            vlo = g[j, pl.ds(0, 16)]
            vhi = g[j, pl.ds(16, 16)]
            plsc.store_scatter(st, [cbv, i1_lo, i2, nnv], vlo)
            plsc.store_scatter(st, [cbv, i1_hi, i2, nnv], vhi)
        return carry

    lax.fori_loop(0, _CPU * 16, tj, 0)


def _body(tok_hbm, table_hbm, out_hbm, idx_a, idx_b, g_a, g_b, st,
          gs_a, gs_b, st_sem):
    cid = lax.axis_index("c")
    sid = lax.axis_index("s")
    wid = sid * 2 + cid

    def fire(t, idx_v, g, gsem):
        u = wid * _PER_W + t
        s = u // _CBLK
        cb = u % _CBLK
        pltpu.sync_copy(tok_hbm.at[s, pl.ds(cb * _CPU, _CPU)], idx_v)
        for c in range(_CPU):
            pltpu.async_copy(table_hbm.at[idx_v.at[c]],
                             g.at[pl.ds(c * 128, 128)], gsem)

    def drain(idx_v, g, gsem):
        for c in range(_CPU):
            pltpu.make_async_copy(table_hbm.at[idx_v.at[c]],
                                  g.at[pl.ds(c * 128, 128)], gsem).wait()

    def store(t):
        u = wid * _PER_W + t
        s = u // _CBLK
        cb = u % _CBLK
        for c in range(_CPU):
            for r in range(4):
                pltpu.async_copy(st.at[c, r, :, pl.ds(0, 128)],
                                 out_hbm.at[s, r, cb * _CPU + c], st_sem)

    def wait_store():
        for c in range(_CPU):
            for r in range(4):
                pltpu.make_async_copy(st.at[c, r, :, pl.ds(0, 128)],
                                      out_hbm.at[0, r, 0], st_sem).wait()

    fire(0, idx_a, g_a, gs_a)

    def step(p, carry):
        t0 = 2 * p
        fire(t0 + 1, idx_b, g_b, gs_b)
        drain(idx_a, g_a, gs_a)
        @pl.when(p > 0)
        def _():
            wait_store()
        _transpose_unit(g_a, st)
        store(t0)
        @pl.when(p < _PER_W // 2 - 1)
        def _():
            fire(t0 + 2, idx_a, g_a, gs_a)
        drain(idx_b, g_b, gs_b)
        wait_store()
        _transpose_unit(g_b, st)
        store(t0 + 1)
        return carry

    lax.fori_loop(0, _PER_W // 2, step, 0)
    wait_store()


@jax.jit
def _embed(tok3, weight):
    mesh = plsc.VectorSubcoreMesh(core_axis_name="c", subcore_axis_name="s")
    kern = functools.partial(
        pl.kernel,
        mesh=mesh,
        out_type=jax.ShapeDtypeStruct((_S, 4, 128, 8, 128), jnp.float32),
        scratch_types=[
            pltpu.VMEM((_CPU, 128), jnp.int32),
            pltpu.VMEM((_CPU, 128), jnp.int32),
            pltpu.VMEM((_UN, _D), jnp.float32),
            pltpu.VMEM((_UN, _D), jnp.float32),
            pltpu.VMEM((_CPU, 4, 8, 133), jnp.float32),
            pltpu.SemaphoreType.DMA,
            pltpu.SemaphoreType.DMA,
            pltpu.SemaphoreType.DMA,
        ],
        compiler_params=pltpu.CompilerParams(
            use_tc_tiling_on_sc=False, needs_layout_passes=False),
    )(_body)
    return kern(tok3, weight)


def kernel(token_ids, weight):
    tok3 = token_ids.T.reshape(_S, 128, 128).astype(jnp.int32)
    out5 = _embed(tok3, weight)
    return out5.transpose((2, 4, 0, 1, 3)).reshape(_B, _S, _D)


# speedup vs baseline: 2.4588x; 1.0002x over previous
import functools

import jax
import jax.numpy as jnp
from jax import lax
from jax.experimental import pallas as pl
from jax.experimental.pallas import tpu as pltpu
from jax.experimental.pallas import tpu_sc as plsc

_B, _S = 16384, 50
_D = 32
_NW = 32
_CPU = 4
_UN = _CPU * 128
_NUNITS = _S * (128 // _CPU)
_PER_W = _NUNITS // _NW
_CBLK = 128 // _CPU


def _transpose_unit(g, st):
    d16 = lax.broadcasted_iota(jnp.int32, (16,), 0)
    i1_lo = d16 // 8
    i1_hi = i1_lo + 2
    i2 = d16 % 8

    def tj(jo, carry):
        cb = jo // 16
        k8 = jo % 16
        row0 = cb * 128 + k8 * 8
        cbv = jnp.full((16,), cb, jnp.int32)
        for u in range(8):
            j = row0 + u
            nnv = jnp.full((16,), k8 * 8 + u, jnp.int32)
            vlo = g[j, pl.ds(0, 16)]
            vhi = g[j, pl.ds(16, 16)]
            plsc.store_scatter(st, [cbv, i1_lo, i2, nnv], vlo)
            plsc.store_scatter(st, [cbv, i1_hi, i2, nnv], vhi)
        return carry

    lax.fori_loop(0, _CPU * 16, tj, 0)


def _body(tok_hbm, table_hbm, out_hbm, idx_a, idx_b, g_a, g_b, st_a, st_b,
          gs_a, gs_b, ss_a, ss_b):
    cid = lax.axis_index("c")
    sid = lax.axis_index("s")
    wid = sid * 2 + cid

    def fire(t, idx_v, g, gsem):
        u = wid * _PER_W + t
        s = u // _CBLK
        cb = u % _CBLK
        pltpu.sync_copy(tok_hbm.at[s, pl.ds(cb * _UN, _UN)], idx_v)
        pltpu.async_copy(table_hbm.at[idx_v], g, gsem)

    def drain(idx_v, g, gsem):
        pltpu.make_async_copy(table_hbm.at[idx_v], g, gsem).wait()

    def store(t, st, ssem):
        u = wid * _PER_W + t
        s = u // _CBLK
        cb = u % _CBLK
        for c in range(_CPU):
            for r in range(4):
                pltpu.async_copy(st.at[c, r, :, pl.ds(0, 128)],
                                 out_hbm.at[s, r, cb * _CPU + c], ssem)

    def wait_store(st, ssem):
        for c in range(_CPU):
            for r in range(4):
                pltpu.make_async_copy(st.at[c, r, :, pl.ds(0, 128)],
                                      out_hbm.at[0, r, 0], ssem).wait()

    fire(0, idx_a, g_a, gs_a)

    def step(p, carry):
        t0 = 2 * p
        fire(t0 + 1, idx_b, g_b, gs_b)
        drain(idx_a, g_a, gs_a)
        @pl.when(p > 0)
        def _():
            wait_store(st_a, ss_a)
        _transpose_unit(g_a, st_a)
        store(t0, st_a, ss_a)
        @pl.when(p < _PER_W // 2 - 1)
        def _():
            fire(t0 + 2, idx_a, g_a, gs_a)
        drain(idx_b, g_b, gs_b)
        @pl.when(p > 0)
        def _():
            wait_store(st_b, ss_b)
        _transpose_unit(g_b, st_b)
        store(t0 + 1, st_b, ss_b)
        return carry

    lax.fori_loop(0, _PER_W // 2, step, 0)
    wait_store(st_a, ss_a)
    wait_store(st_b, ss_b)


@jax.jit
def _embed(tok2, weight):
    mesh = plsc.VectorSubcoreMesh(core_axis_name="c", subcore_axis_name="s")
    kern = functools.partial(
        pl.kernel,
        mesh=mesh,
        out_type=jax.ShapeDtypeStruct((_S, 4, 128, 8, 128), jnp.float32),
        scratch_types=[
            pltpu.VMEM((_UN,), jnp.int32),
            pltpu.VMEM((_UN,), jnp.int32),
            pltpu.VMEM((_UN, _D), jnp.float32),
            pltpu.VMEM((_UN, _D), jnp.float32),
            pltpu.VMEM((_CPU, 4, 8, 133), jnp.float32),
            pltpu.VMEM((_CPU, 4, 8, 133), jnp.float32),
            pltpu.SemaphoreType.DMA,
            pltpu.SemaphoreType.DMA,
            pltpu.SemaphoreType.DMA,
            pltpu.SemaphoreType.DMA,
        ],
        compiler_params=pltpu.CompilerParams(
            use_tc_tiling_on_sc=False, needs_layout_passes=False),
    )(_body)
    return kern(tok2, weight)


def kernel(token_ids, weight):
    tok2 = token_ids.T.astype(jnp.int32)
    out5 = _embed(tok2, weight)
    return out5.transpose((2, 4, 0, 1, 3)).reshape(_B, _S, _D)


# speedup vs baseline: 2.5684x; 1.0446x over previous
import functools

import jax
import jax.numpy as jnp
from jax import lax
from jax.experimental import pallas as pl
from jax.experimental.pallas import tpu as pltpu
from jax.experimental.pallas import tpu_sc as plsc

_B, _S = 16384, 50
_D = 32
_NW = 32
_CPU = 4
_UN = _CPU * 128


def _transpose_unit(g, st):
    d16 = lax.broadcasted_iota(jnp.int32, (16,), 0)
    i1_lo = d16 // 8
    i1_hi = i1_lo + 2
    i2 = d16 % 8

    def tj(jo, carry):
        cb = jo // 16
        k8 = jo % 16
        row0 = cb * 128 + k8 * 8
        cbv = jnp.full((16,), cb, jnp.int32)
        for u in range(8):
            j = row0 + u
            nnv = jnp.full((16,), k8 * 8 + u, jnp.int32)
            vlo = g[j, pl.ds(0, 16)]
            vhi = g[j, pl.ds(16, 16)]
            plsc.store_scatter(st, [cbv, i1_lo, i2, nnv], vlo)
            plsc.store_scatter(st, [cbv, i1_hi, i2, nnv], vhi)
        return carry

    lax.fori_loop(0, _CPU * 16, tj, 0)


def _body(tok_hbm, table_hbm, out_hbm, tokv, idx_a, idx_b, g_a, g_b,
          st_a, st_b, gs_a, gs_b, ss_a, ss_b):
    cid = lax.axis_index("c")
    sid = lax.axis_index("s")
    wid = sid * 2 + cid
    n0 = wid * _UN
    c0 = wid * _CPU
    pltpu.sync_copy(tok_hbm.at[pl.ds(n0, _UN)], tokv)
    iota = lax.broadcasted_iota(jnp.int32, (16,), 0)

    def fire(s, idx_v, g, gsem):
        sv = jnp.full((16,), s, jnp.int32)
        for j in range(_UN // 16):
            rows = j * 16 + iota
            idx_v[pl.ds(j * 16, 16)] = plsc.load_gather(tokv, [rows, sv])
        pltpu.async_copy(table_hbm.at[idx_v], g, gsem)

    def drain(idx_v, g, gsem):
        pltpu.make_async_copy(table_hbm.at[idx_v], g, gsem).wait()

    def store(s, st, ssem):
        for c in range(_CPU):
            for r in range(4):
                pltpu.async_copy(st.at[c, r, :, pl.ds(0, 128)],
                                 out_hbm.at[s, r, c0 + c], ssem)

    def wait_store(st, ssem):
        for c in range(_CPU):
            for r in range(4):
                pltpu.make_async_copy(st.at[c, r, :, pl.ds(0, 128)],
                                      out_hbm.at[0, r, 0], ssem).wait()

    fire(0, idx_a, g_a, gs_a)

    def step(p, carry):
        s0 = 2 * p
        fire(s0 + 1, idx_b, g_b, gs_b)
        drain(idx_a, g_a, gs_a)
        @pl.when(p > 0)
        def _():
            wait_store(st_a, ss_a)
        _transpose_unit(g_a, st_a)
        store(s0, st_a, ss_a)
        @pl.when(p < _S // 2 - 1)
        def _():
            fire(s0 + 2, idx_a, g_a, gs_a)
        drain(idx_b, g_b, gs_b)
        @pl.when(p > 0)
        def _():
            wait_store(st_b, ss_b)
        _transpose_unit(g_b, st_b)
        store(s0 + 1, st_b, ss_b)
        return carry

    lax.fori_loop(0, _S // 2, step, 0)
    wait_store(st_a, ss_a)
    wait_store(st_b, ss_b)


@jax.jit
def _embed(tok2, weight):
    mesh = plsc.VectorSubcoreMesh(core_axis_name="c", subcore_axis_name="s")
    kern = functools.partial(
        pl.kernel,
        mesh=mesh,
        out_type=jax.ShapeDtypeStruct((_S, 4, 128, 8, 128), jnp.float32),
        scratch_types=[
            pltpu.VMEM((_UN, _S), jnp.int32),
            pltpu.VMEM((_UN,), jnp.int32),
            pltpu.VMEM((_UN,), jnp.int32),
            pltpu.VMEM((_UN, _D), jnp.float32),
            pltpu.VMEM((_UN, _D), jnp.float32),
            pltpu.VMEM((_CPU, 4, 8, 133), jnp.float32),
            pltpu.VMEM((_CPU, 4, 8, 133), jnp.float32),
            pltpu.SemaphoreType.DMA,
            pltpu.SemaphoreType.DMA,
            pltpu.SemaphoreType.DMA,
            pltpu.SemaphoreType.DMA,
        ],
        compiler_params=pltpu.CompilerParams(
            use_tc_tiling_on_sc=False, needs_layout_passes=False),
    )(_body)
    return kern(tok2, weight)


def kernel(token_ids, weight):
    out5 = _embed(token_ids.astype(jnp.int32), weight)
    return out5.transpose((2, 4, 0, 1, 3)).reshape(_B, _S, _D)
